# Initial kernel scaffold; baseline (speedup 1.0000x reference)
import jax
import jax.numpy as jnp
from jax.experimental import pallas as pl


def kernel(rel_bias, context_sq_bias, sq_context_bias, context_context_bias):
    raise NotImplementedError("write your pallas kernel here")


# speedup vs baseline: 1.6722x; 1.6722x over previous
import functools

import numpy as np
import jax
import jax.numpy as jnp
from jax import lax
from jax.experimental import pallas as pl
from jax.experimental.pallas import tpu as pltpu
from jax.experimental.pallas import tpu_sc as plsc

_H = 32
_C = 3
_S = 67
_REL = 15 * 15
_CSB = 3 * 64
_SCB = 64 * 3
_CCB = 3 * 3
_TBL = _REL + _CSB + _SCB + _CCB
_TBL_PAD = 624
_OUT = _S * _S
_NVEC = 281
_OUT_PAD = _NVEC * 16


def _build_index_map() -> np.ndarray:
    idx = np.zeros(_OUT_PAD, np.int32)
    for r in range(_S):
        for c in range(_S):
            d = r * _S + c
            if r < _C and c < _C:
                idx[d] = _REL + _CSB + _SCB + r * _C + c
            elif r < _C:
                idx[d] = _REL + r * 64 + (c - _C)
            elif c < _C:
                idx[d] = _REL + _CSB + (r - _C) * _C + c
            else:
                i, j = r - _C, c - _C
                dr = i // 8 - j // 8 + 7
                df = i % 8 - j % 8 + 7
                idx[d] = dr * 15 + df
    return idx


_IDX_MAP = _build_index_map()


@functools.cache
def _gather_heads_fn():
    mesh = plsc.VectorSubcoreMesh(core_axis_name="c", subcore_axis_name="s")

    @functools.partial(
        pl.kernel,
        out_type=jax.ShapeDtypeStruct((_H, _OUT_PAD), jnp.float32),
        mesh=mesh,
        scratch_types=[
            pltpu.VMEM((_TBL_PAD,), jnp.float32),
            pltpu.VMEM((_OUT_PAD,), jnp.int32),
            pltpu.VMEM((_OUT_PAD,), jnp.float32),
        ],
        compiler_params=pltpu.CompilerParams(needs_layout_passes=False),
    )
    def _gather_heads(tbl_hbm, idx_hbm, out_hbm, tbl_v, idx_v, out_v):
        num_cores = lax.axis_size("c")
        h = lax.axis_index("s") * num_cores + lax.axis_index("c")
        pltpu.sync_copy(idx_hbm, idx_v)
        pltpu.sync_copy(tbl_hbm.at[h], tbl_v)

        def step(k, carry):
            base = pl.ds(k * 16, 16)
            out_v[base] = plsc.load_gather(tbl_v, [idx_v[base]])
            return carry

        lax.fori_loop(0, _NVEC, step, 0)
        pltpu.sync_copy(out_v, out_hbm.at[h])

    return _gather_heads


def kernel(rel_bias, context_sq_bias, sq_context_bias, context_context_bias):
    H = rel_bias.shape[0]
    tbl = jnp.concatenate(
        [
            rel_bias.reshape(H, _REL),
            context_sq_bias.reshape(H, _CSB),
            sq_context_bias.reshape(H, _SCB),
            context_context_bias.reshape(H, _CCB),
            jnp.zeros((H, _TBL_PAD - _TBL), rel_bias.dtype),
        ],
        axis=1,
    )
    idx = jnp.asarray(_IDX_MAP)
    out = _gather_heads_fn()(tbl, idx)
    return out[:, :_OUT].reshape(H, _S, _S)
